# baseline (speedup 1.0000x reference)
import functools

import jax
import jax.numpy as jnp
from jax import lax
from jax.experimental import pallas as pl
from jax.experimental.pallas import tpu as pltpu
from jax.experimental.pallas import tpu_sc as plsc

_NBUF = 4
_LOOKAHEAD = 2


def _sc_embed(x3, table, pos, *, NW, n_ch, CH, T, D, L):
    NC = 2
    mesh = plsc.VectorSubcoreMesh(core_axis_name="c", subcore_axis_name="s")
    per_w = T // NW

    @functools.partial(
        pl.kernel,
        mesh=mesh,
        out_type=jax.ShapeDtypeStruct((T, D), jnp.float32),
        scratch_types=(
            [pltpu.VMEM((n_ch, CH), jnp.int32)]
            + [pltpu.VMEM((CH, D), jnp.float32) for _ in range(_NBUF)]
            + [pltpu.VMEM((L, D), jnp.float32)]
            + [pltpu.SemaphoreType.DMA for _ in range(2 * _NBUF)]
        ),
    )
    def k(x_hbm, tab_hbm, pos_hbm, out_hbm, idx_v, *rest):
        rows = rest[:_NBUF]
        pos_v = rest[_NBUF]
        gsem = rest[_NBUF + 1:2 * _NBUF + 1]
        osem = rest[2 * _NBUF + 1:]
        c = lax.axis_index("c")
        s = lax.axis_index("s")
        wid = s * NC + c
        pltpu.sync_copy(pos_hbm, pos_v)
        pltpu.sync_copy(x_hbm.at[wid], idx_v)
        for b in range(_LOOKAHEAD):
            pltpu.async_copy(tab_hbm.at[idx_v.at[b]], rows[b], gsem[b])

        def group(Gi, carry):
            G = Gi * _NBUF
            for b in range(_NBUF):
                g = G + b
                b2 = (b + _LOOKAHEAD) % _NBUF

                @pl.when(g + _LOOKAHEAD < n_ch)
                def _issue():
                    @pl.when(g >= _NBUF - _LOOKAHEAD)
                    def _drain():
                        pltpu.make_async_copy(
                            rows[b2], out_hbm.at[pl.ds(wid * per_w, CH)], osem[b2]
                        ).wait()

                    pltpu.async_copy(
                        tab_hbm.at[idx_v.at[g + _LOOKAHEAD]], rows[b2], gsem[b2]
                    )

                pltpu.make_async_copy(
                    tab_hbm.at[idx_v.at[g]], rows[b], gsem[b]
                ).wait()

                off = (g * CH) % L
                split = jnp.minimum(L - off, CH)

                def add_lo(l, c2, _b=b):
                    for j in range(D // 16):
                        sl = pl.ds(j * 16, 16)
                        rows[_b][l, sl] = rows[_b][l, sl] + pos_v[off + l, sl]
                    return c2

                def add_hi(l, c2, _b=b):
                    for j in range(D // 16):
                        sl = pl.ds(j * 16, 16)
                        rows[_b][l, sl] = rows[_b][l, sl] + pos_v[off - L + l, sl]
                    return c2

                lax.fori_loop(0, split, add_lo, 0)
                lax.fori_loop(split, CH, add_hi, 0)
                pltpu.async_copy(
                    rows[b], out_hbm.at[pl.ds(wid * per_w + g * CH, CH)], osem[b]
                )
            return carry

        lax.fori_loop(0, n_ch // _NBUF, group, 0)
        for b in range(_NBUF):
            pltpu.make_async_copy(
                rows[b], out_hbm.at[pl.ds(wid * per_w, CH)], osem[b]
            ).wait()

    return k(x3, table, pos)


def kernel(x, token_embedding, positional_embedding):
    B, L = x.shape
    V, D = token_embedding.shape
    T = B * L
    NW = 32
    CH = 128
    per_w = T // NW
    n_ch = per_w // CH
    x3 = x.reshape(NW, n_ch, CH).astype(jnp.int32)
    out = _sc_embed(
        x3, token_embedding, positional_embedding,
        NW=NW, n_ch=n_ch, CH=CH, T=T, D=D, L=L,
    )
    return out.reshape(B, L, D)


# speedup vs baseline: 2.9694x; 2.9694x over previous
import functools

import jax
import jax.numpy as jnp
from jax import lax
from jax.experimental import pallas as pl
from jax.experimental.pallas import tpu as pltpu
from jax.experimental.pallas import tpu_sc as plsc

_NBUF = 4
_LOOKAHEAD = 2


def _sc_embed(x4, table, pos, *, NW, n_ch, CH, T, D, L):
    NC = 2
    mesh = plsc.VectorSubcoreMesh(core_axis_name="c", subcore_axis_name="s")
    per_w = T // NW
    H = CH // 2

    @functools.partial(
        pl.kernel,
        mesh=mesh,
        out_type=jax.ShapeDtypeStruct((T, D), jnp.float32),
        scratch_types=(
            [pltpu.VMEM((_NBUF, 2, H), jnp.int32)]
            + [pltpu.VMEM((CH, D), jnp.float32) for _ in range(_NBUF)]
            + [pltpu.VMEM((L, D), jnp.float32)]
            + [pltpu.SemaphoreType.DMA for _ in range(2 * _NBUF)]
        ),
    )
    def k(x_hbm, tab_hbm, pos_hbm, out_hbm, idx_v, *rest):
        rows = rest[:_NBUF]
        pos_v = rest[_NBUF]
        gsem = rest[_NBUF + 1:2 * _NBUF + 1]
        osem = rest[2 * _NBUF + 1:]
        c = lax.axis_index("c")
        s = lax.axis_index("s")
        wid = s * NC + c
        pltpu.sync_copy(pos_hbm, pos_v)

        def start_gather(g, b):
            pltpu.sync_copy(x_hbm.at[wid, g], idx_v.at[b])
            pltpu.async_copy(
                tab_hbm.at[idx_v.at[b, 0]], rows[b].at[pl.ds(0, H)], gsem[b]
            )
            pltpu.async_copy(
                tab_hbm.at[idx_v.at[b, 1]], rows[b].at[pl.ds(H, H)], gsem[b]
            )

        def wait_gather(b):
            pltpu.make_async_copy(
                tab_hbm.at[idx_v.at[b, 0]], rows[b].at[pl.ds(0, H)], gsem[b]
            ).wait()
            pltpu.make_async_copy(
                tab_hbm.at[idx_v.at[b, 1]], rows[b].at[pl.ds(H, H)], gsem[b]
            ).wait()

        for b in range(_LOOKAHEAD):
            start_gather(b, b)

        def group(Gi, carry):
            G = Gi * _NBUF
            for b in range(_NBUF):
                g = G + b
                b2 = (b + _LOOKAHEAD) % _NBUF

                @pl.when(g + _LOOKAHEAD < n_ch)
                def _issue():
                    @pl.when(g >= _NBUF - _LOOKAHEAD)
                    def _drain():
                        pltpu.make_async_copy(
                            rows[b2], out_hbm.at[pl.ds(wid * per_w, CH)], osem[b2]
                        ).wait()

                    start_gather(g + _LOOKAHEAD, b2)

                wait_gather(b)

                def add_row(l, c2, _b=b):
                    for j in range(D // 16):
                        sl = pl.ds(j * 16, 16)
                        rows[_b][l, sl] = rows[_b][l, sl] + pos_v[l, sl]
                    return c2

                lax.fori_loop(0, CH, add_row, 0)
                pltpu.async_copy(
                    rows[b], out_hbm.at[pl.ds(wid * per_w + g * CH, CH)], osem[b]
                )
            return carry

        lax.fori_loop(0, n_ch // _NBUF, group, 0)
        for b in range(_NBUF):
            pltpu.make_async_copy(
                rows[b], out_hbm.at[pl.ds(wid * per_w, CH)], osem[b]
            ).wait()

    return k(x4, table, pos)


def kernel(x, token_embedding, positional_embedding):
    B, L = x.shape
    V, D = token_embedding.shape
    T = B * L
    NW = 32
    CH = L
    per_w = T // NW
    n_ch = per_w // CH
    x4 = x.reshape(NW, n_ch, 2, CH // 2).astype(jnp.int32)
    out = _sc_embed(
        x4, token_embedding, positional_embedding,
        NW=NW, n_ch=n_ch, CH=CH, T=T, D=D, L=L,
    )
    return out.reshape(B, L, D)


# speedup vs baseline: 3.2724x; 1.1020x over previous
import functools

import jax
import jax.numpy as jnp
from jax import lax
from jax.experimental import pallas as pl
from jax.experimental.pallas import tpu as pltpu
from jax.experimental.pallas import tpu_sc as plsc

_NBUF = 4
_GLA = 2
_ILA = 3


def _sc_embed(x4, table, pos, *, NW, n_ch, CH, T, D, L):
    NC = 2
    mesh = plsc.VectorSubcoreMesh(core_axis_name="c", subcore_axis_name="s")
    per_w = T // NW
    H = CH // 2

    @functools.partial(
        pl.kernel,
        mesh=mesh,
        out_type=jax.ShapeDtypeStruct((T, D), jnp.float32),
        scratch_types=(
            [pltpu.VMEM((_NBUF, 2, H), jnp.int32)]
            + [pltpu.VMEM((CH, D), jnp.float32) for _ in range(_NBUF)]
            + [pltpu.VMEM((L, D), jnp.float32)]
            + [pltpu.SemaphoreType.DMA for _ in range(3 * _NBUF)]
        ),
    )
    def k(x_hbm, tab_hbm, pos_hbm, out_hbm, idx_v, *rest):
        rows = rest[:_NBUF]
        pos_v = rest[_NBUF]
        gsem = rest[_NBUF + 1:2 * _NBUF + 1]
        osem = rest[2 * _NBUF + 1:3 * _NBUF + 1]
        isem = rest[3 * _NBUF + 1:]
        c = lax.axis_index("c")
        s = lax.axis_index("s")
        wid = s * NC + c
        pltpu.sync_copy(pos_hbm, pos_v)

        def issue_idx(g, b):
            pltpu.async_copy(x_hbm.at[wid, g], idx_v.at[b], isem[b])

        def wait_idx(b):
            pltpu.make_async_copy(x_hbm.at[wid, 0], idx_v.at[b], isem[b]).wait()

        def start_gather(b):
            pltpu.async_copy(
                tab_hbm.at[idx_v.at[b, 0]], rows[b].at[pl.ds(0, H)], gsem[b]
            )
            pltpu.async_copy(
                tab_hbm.at[idx_v.at[b, 1]], rows[b].at[pl.ds(H, H)], gsem[b]
            )

        def wait_gather(b):
            pltpu.make_async_copy(
                tab_hbm.at[idx_v.at[b, 0]], rows[b].at[pl.ds(0, H)], gsem[b]
            ).wait()
            pltpu.make_async_copy(
                tab_hbm.at[idx_v.at[b, 1]], rows[b].at[pl.ds(H, H)], gsem[b]
            ).wait()

        for b in range(_ILA):
            issue_idx(b, b)
        for b in range(_GLA):
            wait_idx(b)
            start_gather(b)

        def group(Gi, carry):
            G = Gi * _NBUF
            for b in range(_NBUF):
                g = G + b
                b2 = (b + _GLA) % _NBUF
                b3 = (b + _ILA) % _NBUF

                @pl.when(g + _GLA < n_ch)
                def _issue():
                    @pl.when(g >= _NBUF - _GLA)
                    def _drain():
                        pltpu.make_async_copy(
                            rows[b2], out_hbm.at[pl.ds(wid * per_w, CH)], osem[b2]
                        ).wait()

                    wait_idx(b2)
                    start_gather(b2)

                @pl.when(g + _ILA < n_ch)
                def _prefetch():
                    issue_idx(g + _ILA, b3)

                wait_gather(b)

                def add_row(l, c2, _b=b):
                    for j in range(D // 16):
                        sl = pl.ds(j * 16, 16)
                        rows[_b][l, sl] = rows[_b][l, sl] + pos_v[l, sl]
                    return c2

                lax.fori_loop(0, CH, add_row, 0)
                pltpu.async_copy(
                    rows[b], out_hbm.at[pl.ds(wid * per_w + g * CH, CH)], osem[b]
                )
            return carry

        lax.fori_loop(0, n_ch // _NBUF, group, 0)
        for b in range(_NBUF):
            pltpu.make_async_copy(
                rows[b], out_hbm.at[pl.ds(wid * per_w, CH)], osem[b]
            ).wait()

    return k(x4, table, pos)


def kernel(x, token_embedding, positional_embedding):
    B, L = x.shape
    V, D = token_embedding.shape
    T = B * L
    NW = 32
    CH = L
    per_w = T // NW
    n_ch = per_w // CH
    x4 = x.reshape(NW, n_ch, 2, CH // 2).astype(jnp.int32)
    out = _sc_embed(
        x4, token_embedding, positional_embedding,
        NW=NW, n_ch=n_ch, CH=CH, T=T, D=D, L=L,
    )
    return out.reshape(B, L, D)
